# baseline (speedup 1.0000x reference)
import functools

import jax
import jax.numpy as jnp
from jax import lax
from jax.experimental import pallas as pl
from jax.experimental.pallas import tpu as pltpu
from jax.experimental.pallas import tpu_sc as plsc

VOCAB = 1000000
DIM = 64
BATCH = 16384

NC = 2
NS = 16
NW = NC * NS
B_PER_W = BATCH // NW
CHUNK = 128
NCHUNK = B_PER_W // CHUNK


def _sc_gather(user2d, item2d, embed_user, embed_item):
  mesh = plsc.VectorSubcoreMesh(
      core_axis_name="c", subcore_axis_name="s",
      num_cores=NC, num_subcores=NS)

  @functools.partial(
      pl.kernel,
      out_type=(
          jax.ShapeDtypeStruct((BATCH, DIM), jnp.float32),
          jax.ShapeDtypeStruct((BATCH, DIM), jnp.float32),
      ),
      mesh=mesh,
      compiler_params=pltpu.CompilerParams(use_tc_tiling_on_sc=False),
      scratch_types=[
          pltpu.VMEM((NCHUNK, CHUNK), jnp.int32),
          pltpu.VMEM((NCHUNK, CHUNK), jnp.int32),
          pltpu.VMEM((B_PER_W, DIM), jnp.float32),
          pltpu.VMEM((B_PER_W, DIM), jnp.float32),
          pltpu.SemaphoreType.DMA,
      ],
  )
  def k(u_hbm, i_hbm, eu_hbm, ei_hbm, vu_out, vi_out,
        idx_u, idx_i, rows_u, rows_i, sem):
    wid = lax.axis_index("s") * NC + lax.axis_index("c")
    pltpu.sync_copy(u_hbm.at[pl.ds(wid * NCHUNK, NCHUNK)], idx_u)
    pltpu.sync_copy(i_hbm.at[pl.ds(wid * NCHUNK, NCHUNK)], idx_i)
    copies = []
    for j in range(NCHUNK):
      dst = pl.ds(j * CHUNK, CHUNK)
      copies.append(pltpu.async_copy(eu_hbm.at[idx_u.at[j]],
                                     rows_u.at[dst], sem))
      copies.append(pltpu.async_copy(ei_hbm.at[idx_i.at[j]],
                                     rows_i.at[dst], sem))
    for c in copies:
      c.wait()
    base = wid * B_PER_W
    pltpu.sync_copy(rows_u, vu_out.at[pl.ds(base, B_PER_W)])
    pltpu.sync_copy(rows_i, vi_out.at[pl.ds(base, B_PER_W)])

  return k(user2d, item2d, embed_user, embed_item)


def _tc_body(u_ref, i_ref, w1u_ref, w1i_ref, b1_ref, w2_ref, b2_ref,
             w3_ref, b3_ref, out_ref):
  u = u_ref[...]
  v = i_ref[...]
  pred = jnp.sum(u * v, axis=1)
  h = jnp.maximum(
      u @ w1u_ref[...] + v @ w1i_ref[...] + b1_ref[...], 0.0)
  h = jnp.maximum(h @ w2_ref[...] + b2_ref[...], 0.0)
  noise = jnp.maximum(h @ w3_ref[...] + b3_ref[...], 0.0)
  out_ref[...] = pred + noise[:, 0]


def kernel(user, item, embed_user, embed_item, W1, b1, W2, b2, W3, b3):
  user2d = user.astype(jnp.int32).reshape(BATCH // CHUNK, CHUNK)
  item2d = item.astype(jnp.int32).reshape(BATCH // CHUNK, CHUNK)
  vec_u, vec_i = _sc_gather(user2d, item2d, embed_user, embed_item)

  w1u = W1[:DIM]
  w1i = W1[DIM:]
  out = pl.pallas_call(
      _tc_body,
      out_shape=jax.ShapeDtypeStruct((BATCH,), jnp.float32),
  )(vec_u, vec_i, w1u, w1i, b1, W2, b2, W3, b3)
  return out
